# speedup vs baseline: 1.5007x; 1.5007x over previous
import jax
import jax.numpy as jnp
from jax.experimental import pallas as pl


def kernel(x, W):
    b, l, d = x.shape
    BLK_L = 512

    def body(x_ref, w_ref, o_ref):
        o_ref[...] = x_ref[...] + w_ref[...]

    return pl.pallas_call(
        body,
        grid=(l // BLK_L, b),
        in_specs=[
            pl.BlockSpec((1, BLK_L, d), lambda i, j: (j, i, 0)),
            pl.BlockSpec((BLK_L, d), lambda i, j: (i, 0)),
        ],
        out_specs=pl.BlockSpec((1, BLK_L, d), lambda i, j: (j, i, 0)),
        out_shape=jax.ShapeDtypeStruct(x.shape, x.dtype),
    )(x, W)


# speedup vs baseline: 1.6706x; 1.1132x over previous
import jax
import jax.numpy as jnp
from jax.experimental import pallas as pl


def kernel(x, W):
    b, l, d = x.shape
    BLK_L = 1024

    def body(x_ref, w_ref, o_ref):
        o_ref[...] = x_ref[...] + w_ref[...]

    return pl.pallas_call(
        body,
        grid=(l // BLK_L, b),
        in_specs=[
            pl.BlockSpec((1, BLK_L, d), lambda i, j: (j, i, 0)),
            pl.BlockSpec((BLK_L, d), lambda i, j: (i, 0)),
        ],
        out_specs=pl.BlockSpec((1, BLK_L, d), lambda i, j: (j, i, 0)),
        out_shape=jax.ShapeDtypeStruct(x.shape, x.dtype),
    )(x, W)


# speedup vs baseline: 1.7359x; 1.0391x over previous
import jax
import jax.numpy as jnp
from jax.experimental import pallas as pl


def kernel(x, W):
    b, l, d = x.shape
    BLK_L = 2048

    def body(x_ref, w_ref, o_ref):
        o_ref[...] = x_ref[...] + w_ref[...]

    return pl.pallas_call(
        body,
        grid=(l // BLK_L, b),
        in_specs=[
            pl.BlockSpec((1, BLK_L, d), lambda i, j: (j, i, 0)),
            pl.BlockSpec((BLK_L, d), lambda i, j: (i, 0)),
        ],
        out_specs=pl.BlockSpec((1, BLK_L, d), lambda i, j: (j, i, 0)),
        out_shape=jax.ShapeDtypeStruct(x.shape, x.dtype),
    )(x, W)
